# Initial kernel scaffold; baseline (speedup 1.0000x reference)
import functools

import jax
import jax.numpy as jnp
from jax import lax
from jax.experimental import pallas as pl
from jax.experimental.pallas import tpu as pltpu
from jax.experimental.pallas import tpu_sc as plsc

N, E, D = 10000, 320000, 128
NC, NS = 2, 16
NW = NC * NS
CHUNK = 128
CHUNKS = 80
EPW = CHUNK * CHUNKS
E_PAD = EPW * NW
N_PAD = 10112
RPT = N_PAD // NS
_HIGH = lax.Precision.HIGHEST


def _make_sc_agg(with_cnt):
  out_type = [jax.ShapeDtypeStruct((NC, N_PAD, D), jnp.float32)]
  if with_cnt:
    out_type.append(jax.ShapeDtypeStruct((NC, N_PAD, 16), jnp.float32))
  scratch = [
      pltpu.VMEM((CHUNKS, CHUNK), jnp.int32),
      pltpu.VMEM((CHUNKS, CHUNK), jnp.int32),
      pltpu.VMEM((CHUNK, D), jnp.float32),
      pltpu.VMEM((RPT, 16), jnp.float32),
      pltpu.VMEM((CHUNK, 16), jnp.float32),
      pltpu.VMEM_SHARED((N_PAD, D), jnp.float32),
      pltpu.VMEM_SHARED((N_PAD, 16), jnp.float32),
      pltpu.SemaphoreType.DMA,
  ]
  mesh = plsc.VectorSubcoreMesh(core_axis_name="c", subcore_axis_name="s")

  @functools.partial(
      pl.kernel, out_type=tuple(out_type), mesh=mesh,
      scratch_types=scratch)
  def body(xp_hbm, src_hbm, dst_hbm, zeros_hbm, zeros16_hbm, ones_hbm, *rest):
    if with_cnt:
      (acc_out, cnt_out, src_v, dst_v, rowbuf, cbuf, onesbuf,
       acc_sh, cnt_sh, sem) = rest
    else:
      (acc_out, src_v, dst_v, rowbuf, cbuf, onesbuf,
       acc_sh, cnt_sh, sem) = rest
      cnt_out = None
    c = lax.axis_index("c")
    s = lax.axis_index("s")
    wid = s * NC + c

    pltpu.sync_copy(src_hbm.at[wid], src_v)
    pltpu.sync_copy(dst_hbm.at[wid], dst_v)

    pltpu.sync_copy(zeros_hbm, rowbuf)
    base = s * RPT
    off = 0
    for sz in (CHUNK, CHUNK, CHUNK, CHUNK, RPT - 4 * CHUNK):
      pltpu.sync_copy(rowbuf.at[pl.ds(0, sz)],
                      acc_sh.at[pl.ds(base + off, sz)])
      off += sz
    if with_cnt:
      pltpu.sync_copy(zeros16_hbm, cbuf)
      pltpu.sync_copy(cbuf, cnt_sh.at[pl.ds(base, RPT)])
      pltpu.sync_copy(ones_hbm, onesbuf)
    plsc.subcore_barrier()

    def chunk_step(t, carry):
      for k in range(8):
        j = t * 8 + k
        pltpu.async_copy(xp_hbm.at[src_v.at[j]], rowbuf, sem).wait()
        pltpu.sync_copy(rowbuf, acc_sh.at[dst_v.at[j]], add=True)
        if with_cnt:
          pltpu.sync_copy(onesbuf, cnt_sh.at[dst_v.at[j]], add=True)
      return carry
    lax.fori_loop(0, CHUNKS // 8, chunk_step, 0)

    plsc.subcore_barrier()

    off = 0
    for sz in (CHUNK, CHUNK, CHUNK, CHUNK, RPT - 4 * CHUNK):
      pltpu.sync_copy(acc_sh.at[pl.ds(base + off, sz)],
                      rowbuf.at[pl.ds(0, sz)])
      pltpu.sync_copy(rowbuf.at[pl.ds(0, sz)],
                      acc_out.at[c, pl.ds(base + off, sz)])
      off += sz
    if with_cnt:
      pltpu.sync_copy(cnt_sh.at[pl.ds(base, RPT)], cbuf)
      pltpu.sync_copy(cbuf, cnt_out.at[c, pl.ds(base, RPT)])

  return body


_sc_agg_cnt = _make_sc_agg(True)
_sc_agg = _make_sc_agg(False)


_BLK = 632


def _dot(a, b):
  return jnp.dot(a, b, preferred_element_type=jnp.float32, precision=_HIGH)


def _proj_body(x_ref, w_ref, b_ref, o_ref):
  o_ref[...] = jnp.maximum(_dot(x_ref[...], w_ref[...]) + b_ref[...], 0.0)


def _proj(x, W, b):
  return pl.pallas_call(
      _proj_body,
      grid=(N_PAD // _BLK,),
      in_specs=[
          pl.BlockSpec((_BLK, D), lambda i: (i, 0)),
          pl.BlockSpec((D, D), lambda i: (0, 0)),
          pl.BlockSpec((D,), lambda i: (0,)),
      ],
      out_specs=pl.BlockSpec((_BLK, D), lambda i: (i, 0)),
      out_shape=jax.ShapeDtypeStruct((N_PAD, D), jnp.float32),
  )(x, W, b)


def _mean_agg(acc_ref, cnt_ref):
  ssum = acc_ref[0] + acc_ref[1]
  cnt = jnp.sum(cnt_ref[0], axis=-1) + jnp.sum(cnt_ref[1], axis=-1)
  return ssum / jnp.clip(cnt, 1.0, None)[:, None]


def _combine1_body(acc_ref, cnt_ref, x_ref, wl_ref, bl_ref, wr_ref,
                   wp2_ref, bp2_ref, h_ref, xp2_ref):
  agg = _mean_agg(acc_ref, cnt_ref)
  h = jnp.maximum(
      _dot(agg, wl_ref[...]) + bl_ref[...] + _dot(x_ref[...], wr_ref[...]),
      0.0)
  h_ref[...] = h
  xp2_ref[...] = jnp.maximum(_dot(h, wp2_ref[...]) + bp2_ref[...], 0.0)


def _combine1(acc, cnt, x, Wl, bl, Wr, Wp2, bp2):
  return pl.pallas_call(
      _combine1_body,
      grid=(N_PAD // _BLK,),
      in_specs=[
          pl.BlockSpec((NC, _BLK, D), lambda i: (0, i, 0)),
          pl.BlockSpec((NC, _BLK, 16), lambda i: (0, i, 0)),
          pl.BlockSpec((_BLK, D), lambda i: (i, 0)),
          pl.BlockSpec((D, D), lambda i: (0, 0)),
          pl.BlockSpec((D,), lambda i: (0,)),
          pl.BlockSpec((D, D), lambda i: (0, 0)),
          pl.BlockSpec((D, D), lambda i: (0, 0)),
          pl.BlockSpec((D,), lambda i: (0,)),
      ],
      out_specs=[
          pl.BlockSpec((_BLK, D), lambda i: (i, 0)),
          pl.BlockSpec((_BLK, D), lambda i: (i, 0)),
      ],
      out_shape=[
          jax.ShapeDtypeStruct((N_PAD, D), jnp.float32),
          jax.ShapeDtypeStruct((N_PAD, D), jnp.float32),
      ],
  )(acc, cnt, x, Wl, bl, Wr, Wp2, bp2)


def _combine2_body(acc_ref, cnt_ref, h_ref, wl_ref, bl_ref, wr_ref, o_ref):
  agg = _mean_agg(acc_ref, cnt_ref)
  o = (_dot(agg, wl_ref[...]) + bl_ref[...] + _dot(h_ref[...], wr_ref[...]))
  norm = jnp.sqrt(jnp.sum(o * o, axis=-1, keepdims=True))
  o = o / jnp.clip(norm, 1e-12, None)
  o = jnp.maximum(o, 0.0)
  m = jnp.max(o, axis=-1, keepdims=True)
  lse = m + jnp.log(jnp.sum(jnp.exp(o - m), axis=-1, keepdims=True))
  o_ref[...] = o - lse


def _combine2(acc, cnt, h, Wl, bl, Wr):
  return pl.pallas_call(
      _combine2_body,
      grid=(N_PAD // _BLK,),
      in_specs=[
          pl.BlockSpec((NC, _BLK, D), lambda i: (0, i, 0)),
          pl.BlockSpec((NC, _BLK, 16), lambda i: (0, i, 0)),
          pl.BlockSpec((_BLK, D), lambda i: (i, 0)),
          pl.BlockSpec((D, D), lambda i: (0, 0)),
          pl.BlockSpec((D,), lambda i: (0,)),
          pl.BlockSpec((D, D), lambda i: (0, 0)),
      ],
      out_specs=pl.BlockSpec((_BLK, D), lambda i: (i, 0)),
      out_shape=jax.ShapeDtypeStruct((N_PAD, D), jnp.float32),
  )(acc, cnt, h, Wl, bl, Wr)


def kernel(matrix_nodes_features, edge_index, Wp1, bp1, Wl1, bl1, Wr1,
           Wp2, bp2, Wl2, bl2, Wr2):
  x = jnp.pad(matrix_nodes_features, ((0, N_PAD - N), (0, 0)))
  src = jnp.concatenate(
      [edge_index[0], jnp.zeros((E_PAD - E,), jnp.int32)]).reshape(
          NW, CHUNKS, CHUNK)
  dst = jnp.concatenate(
      [edge_index[1], jnp.full((E_PAD - E,), N, jnp.int32)]).reshape(
          NW, CHUNKS, CHUNK)
  zeros = jnp.zeros((CHUNK, D), jnp.float32)
  zeros16 = jnp.zeros((RPT, 16), jnp.float32)
  ones16 = jnp.zeros((CHUNK, 16), jnp.float32).at[:, 0].set(1.0)

  xp1 = _proj(x, Wp1, bp1)
  acc1, cnt = _sc_agg_cnt(xp1, src, dst, zeros, zeros16, ones16)
  h, xp2 = _combine1(acc1, cnt, x, Wl1, bl1, Wr1, Wp2, bp2)
  acc2 = _sc_agg(xp2, src, dst, zeros, zeros16, ones16)
  out = _combine2(acc2, cnt, h, Wl2, bl2, Wr2)
  return out[:N]


# speedup vs baseline: 2.4035x; 2.4035x over previous
import functools

import jax
import jax.numpy as jnp
from jax import lax
from jax.experimental import pallas as pl
from jax.experimental.pallas import tpu as pltpu
from jax.experimental.pallas import tpu_sc as plsc

N, E, D = 10000, 320000, 128
NC, NS = 2, 16
NW = NC * NS
CHUNK = 64
CHUNKS = 160
EPW = CHUNK * CHUNKS
E_PAD = EPW * NW
N_PAD = 10112
RPT = N_PAD // NS
_HIGH = lax.Precision.HIGHEST
_MESH = plsc.VectorSubcoreMesh(core_axis_name="c", subcore_axis_name="s")
_SLICES = tuple([CHUNK] * (RPT // CHUNK) +
                ([RPT % CHUNK] if RPT % CHUNK else []))


def _acc_slice_copy(src_at, dst_at, base):
  off = 0
  for sz in _SLICES:
    pltpu.sync_copy(src_at(base + off, sz), dst_at(base + off, sz))
    off += sz


@functools.partial(
    pl.kernel,
    out_type=(jax.ShapeDtypeStruct((NC, N_PAD, D), jnp.float32),),
    mesh=_MESH,
    scratch_types=[
        pltpu.VMEM((8, CHUNK), jnp.int32),
        pltpu.VMEM((8, CHUNK), jnp.int32),
        pltpu.VMEM((CHUNK, D), jnp.float32),
        pltpu.VMEM_SHARED((N_PAD, D), jnp.float32),
        pltpu.SemaphoreType.DMA,
    ])
def _sc_agg(xp_hbm, src_hbm, dst_hbm, zeros_hbm, acc_out,
            src_v, dst_v, rowbuf, acc_sh, sem):
  c = lax.axis_index("c")
  s = lax.axis_index("s")
  wid = s * NC + c
  base = s * RPT

  pltpu.sync_copy(zeros_hbm, rowbuf)
  _acc_slice_copy(lambda r, sz: rowbuf.at[pl.ds(0, sz)],
                  lambda r, sz: acc_sh.at[pl.ds(r, sz)], base)
  plsc.subcore_barrier()

  def chunk_step(t, carry):
    pltpu.sync_copy(src_hbm.at[wid, pl.ds(t * 8, 8)], src_v)
    pltpu.sync_copy(dst_hbm.at[wid, pl.ds(t * 8, 8)], dst_v)
    for k in range(8):
      pltpu.async_copy(xp_hbm.at[src_v.at[k]], rowbuf, sem).wait()
      pltpu.sync_copy(rowbuf, acc_sh.at[dst_v.at[k]], add=True)
    return carry
  lax.fori_loop(0, CHUNKS // 8, chunk_step, 0)

  plsc.subcore_barrier()

  _acc_slice_copy(lambda r, sz: acc_sh.at[pl.ds(r, sz)],
                  lambda r, sz: rowbuf.at[pl.ds(0, sz)], base)
  _acc_slice_copy(lambda r, sz: rowbuf.at[pl.ds(0, sz)],
                  lambda r, sz: acc_out.at[c, pl.ds(r, sz)], base)


@functools.partial(
    pl.kernel,
    out_type=(jax.ShapeDtypeStruct((NC, N_PAD, D), jnp.float32),),
    mesh=_MESH,
    scratch_types=[
        pltpu.VMEM((8, CHUNK), jnp.int32),
        pltpu.VMEM((CHUNK, D), jnp.float32),
        pltpu.VMEM((CHUNK, D), jnp.float32),
        pltpu.VMEM_SHARED((N_PAD, D), jnp.float32),
    ])
def _sc_cnt(dst_hbm, zeros_hbm, ones_hbm, cnt_out,
            dst_v, onesrows, zbuf, acc_sh):
  c = lax.axis_index("c")
  s = lax.axis_index("s")
  wid = s * NC + c
  base = s * RPT

  pltpu.sync_copy(zeros_hbm, zbuf)
  pltpu.sync_copy(ones_hbm, onesrows)
  _acc_slice_copy(lambda r, sz: zbuf.at[pl.ds(0, sz)],
                  lambda r, sz: acc_sh.at[pl.ds(r, sz)], base)
  plsc.subcore_barrier()

  def chunk_step(t, carry):
    pltpu.sync_copy(dst_hbm.at[wid, pl.ds(t * 8, 8)], dst_v)
    for k in range(8):
      pltpu.sync_copy(onesrows, acc_sh.at[dst_v.at[k]], add=True)
    return carry
  lax.fori_loop(0, CHUNKS // 8, chunk_step, 0)

  plsc.subcore_barrier()

  _acc_slice_copy(lambda r, sz: acc_sh.at[pl.ds(r, sz)],
                  lambda r, sz: zbuf.at[pl.ds(0, sz)], base)
  _acc_slice_copy(lambda r, sz: zbuf.at[pl.ds(0, sz)],
                  lambda r, sz: cnt_out.at[c, pl.ds(r, sz)], base)


_BLK = 632


def _dot(a, b):
  return jnp.dot(a, b, preferred_element_type=jnp.float32, precision=_HIGH)


def _proj_body(x_ref, w_ref, b_ref, o_ref):
  o_ref[...] = jnp.maximum(_dot(x_ref[...], w_ref[...]) + b_ref[...], 0.0)


def _proj(x, W, b):
  return pl.pallas_call(
      _proj_body,
      grid=(N_PAD // _BLK,),
      in_specs=[
          pl.BlockSpec((_BLK, D), lambda i: (i, 0)),
          pl.BlockSpec((D, D), lambda i: (0, 0)),
          pl.BlockSpec((D,), lambda i: (0,)),
      ],
      out_specs=pl.BlockSpec((_BLK, D), lambda i: (i, 0)),
      out_shape=jax.ShapeDtypeStruct((N_PAD, D), jnp.float32),
  )(x, W, b)


def _mean_agg(acc_ref, cnt_ref):
  ssum = acc_ref[0] + acc_ref[1]
  cnt = cnt_ref[0][:, 0:1] + cnt_ref[1][:, 0:1]
  return ssum / jnp.clip(cnt, 1.0, None)


def _combine1_body(acc_ref, cnt_ref, x_ref, wl_ref, bl_ref, wr_ref,
                   wp2_ref, bp2_ref, h_ref, xp2_ref):
  agg = _mean_agg(acc_ref, cnt_ref)
  h = jnp.maximum(
      _dot(agg, wl_ref[...]) + bl_ref[...] + _dot(x_ref[...], wr_ref[...]),
      0.0)
  h_ref[...] = h
  xp2_ref[...] = jnp.maximum(_dot(h, wp2_ref[...]) + bp2_ref[...], 0.0)


def _combine1(acc, cnt, x, Wl, bl, Wr, Wp2, bp2):
  return pl.pallas_call(
      _combine1_body,
      grid=(N_PAD // _BLK,),
      in_specs=[
          pl.BlockSpec((NC, _BLK, D), lambda i: (0, i, 0)),
          pl.BlockSpec((NC, _BLK, D), lambda i: (0, i, 0)),
          pl.BlockSpec((_BLK, D), lambda i: (i, 0)),
          pl.BlockSpec((D, D), lambda i: (0, 0)),
          pl.BlockSpec((D,), lambda i: (0,)),
          pl.BlockSpec((D, D), lambda i: (0, 0)),
          pl.BlockSpec((D, D), lambda i: (0, 0)),
          pl.BlockSpec((D,), lambda i: (0,)),
      ],
      out_specs=[
          pl.BlockSpec((_BLK, D), lambda i: (i, 0)),
          pl.BlockSpec((_BLK, D), lambda i: (i, 0)),
      ],
      out_shape=[
          jax.ShapeDtypeStruct((N_PAD, D), jnp.float32),
          jax.ShapeDtypeStruct((N_PAD, D), jnp.float32),
      ],
  )(acc, cnt, x, Wl, bl, Wr, Wp2, bp2)


def _combine2_body(acc_ref, cnt_ref, h_ref, wl_ref, bl_ref, wr_ref, o_ref):
  agg = _mean_agg(acc_ref, cnt_ref)
  o = (_dot(agg, wl_ref[...]) + bl_ref[...] + _dot(h_ref[...], wr_ref[...]))
  norm = jnp.sqrt(jnp.sum(o * o, axis=-1, keepdims=True))
  o = o / jnp.clip(norm, 1e-12, None)
  o = jnp.maximum(o, 0.0)
  m = jnp.max(o, axis=-1, keepdims=True)
  lse = m + jnp.log(jnp.sum(jnp.exp(o - m), axis=-1, keepdims=True))
  o_ref[...] = o - lse


def _combine2(acc, cnt, h, Wl, bl, Wr):
  return pl.pallas_call(
      _combine2_body,
      grid=(N_PAD // _BLK,),
      in_specs=[
          pl.BlockSpec((NC, _BLK, D), lambda i: (0, i, 0)),
          pl.BlockSpec((NC, _BLK, D), lambda i: (0, i, 0)),
          pl.BlockSpec((_BLK, D), lambda i: (i, 0)),
          pl.BlockSpec((D, D), lambda i: (0, 0)),
          pl.BlockSpec((D,), lambda i: (0,)),
          pl.BlockSpec((D, D), lambda i: (0, 0)),
      ],
      out_specs=pl.BlockSpec((_BLK, D), lambda i: (i, 0)),
      out_shape=jax.ShapeDtypeStruct((N_PAD, D), jnp.float32),
  )(acc, cnt, h, Wl, bl, Wr)


def kernel(matrix_nodes_features, edge_index, Wp1, bp1, Wl1, bl1, Wr1,
           Wp2, bp2, Wl2, bl2, Wr2):
  x = jnp.pad(matrix_nodes_features, ((0, N_PAD - N), (0, 0)))
  src = jnp.concatenate(
      [edge_index[0], jnp.zeros((E_PAD - E,), jnp.int32)]).reshape(
          NW, CHUNKS, CHUNK)
  dst = jnp.concatenate(
      [edge_index[1], jnp.full((E_PAD - E,), N, jnp.int32)]).reshape(
          NW, CHUNKS, CHUNK)
  zeros = jnp.zeros((CHUNK, D), jnp.float32)
  ones = jnp.ones((CHUNK, D), jnp.float32)

  (cnt,) = _sc_cnt(dst, zeros, ones)
  xp1 = _proj(x, Wp1, bp1)
  (acc1,) = _sc_agg(xp1, src, dst, zeros)
  h, xp2 = _combine1(acc1, cnt, x, Wl1, bl1, Wr1, Wp2, bp2)
  (acc2,) = _sc_agg(xp2, src, dst, zeros)
  out = _combine2(acc2, cnt, h, Wl2, bl2, Wr2)
  return out[:N]


# speedup vs baseline: 2.7552x; 1.1463x over previous
import functools

import jax
import jax.numpy as jnp
from jax import lax
from jax.experimental import pallas as pl
from jax.experimental.pallas import tpu as pltpu
from jax.experimental.pallas import tpu_sc as plsc

N, E, D = 10000, 320000, 128
NC, NS = 2, 16
NW = NC * NS
CHUNK = 64
CHUNKS = 160
EPW = CHUNK * CHUNKS
E_PAD = EPW * NW
N_PAD = 10112
RPT = N_PAD // NS
_HIGH = lax.Precision.HIGHEST
_MESH = plsc.VectorSubcoreMesh(core_axis_name="c", subcore_axis_name="s")
_SLICES = tuple([CHUNK] * (RPT // CHUNK) +
                ([RPT % CHUNK] if RPT % CHUNK else []))


def _acc_slice_copy(src_at, dst_at, base):
  off = 0
  for sz in _SLICES:
    pltpu.sync_copy(src_at(base + off, sz), dst_at(base + off, sz))
    off += sz


@functools.partial(
    pl.kernel,
    out_type=(jax.ShapeDtypeStruct((NC, N_PAD, D), jnp.float32),),
    mesh=_MESH,
    scratch_types=[
        pltpu.VMEM((8, CHUNK), jnp.int32),
        pltpu.VMEM((8, CHUNK), jnp.int32),
        pltpu.VMEM((CHUNK, D), jnp.float32),
        pltpu.VMEM((CHUNK, D), jnp.float32),
        pltpu.VMEM_SHARED((N_PAD, D), jnp.float32),
        pltpu.SemaphoreType.DMA,
    ])
def _sc_agg(xp_hbm, src_hbm, dst_hbm, zeros_hbm, acc_out,
            src_v, dst_v, rowbuf, rowbuf2, acc_sh, sem):
  c = lax.axis_index("c")
  s = lax.axis_index("s")
  wid = s * NC + c
  base = s * RPT
  rb = (rowbuf, rowbuf2)

  pltpu.sync_copy(zeros_hbm, rowbuf)
  _acc_slice_copy(lambda r, sz: rowbuf.at[pl.ds(0, sz)],
                  lambda r, sz: acc_sh.at[pl.ds(r, sz)], base)
  plsc.subcore_barrier()

  def chunk_step(t, carry):
    pltpu.sync_copy(src_hbm.at[wid, pl.ds(t * 8, 8)], src_v)
    pltpu.sync_copy(dst_hbm.at[wid, pl.ds(t * 8, 8)], dst_v)
    pltpu.async_copy(xp_hbm.at[src_v.at[0]], rb[0], sem)
    for k in range(8):
      if k + 1 < 8:
        pltpu.async_copy(xp_hbm.at[src_v.at[k + 1]], rb[(k + 1) % 2], sem)
      pltpu.make_async_copy(xp_hbm.at[src_v.at[k]], rb[k % 2], sem).wait()
      pltpu.sync_copy(rb[k % 2], acc_sh.at[dst_v.at[k]], add=True)
    return carry
  lax.fori_loop(0, CHUNKS // 8, chunk_step, 0)

  plsc.subcore_barrier()

  _acc_slice_copy(lambda r, sz: acc_sh.at[pl.ds(r, sz)],
                  lambda r, sz: rowbuf.at[pl.ds(0, sz)], base)
  _acc_slice_copy(lambda r, sz: rowbuf.at[pl.ds(0, sz)],
                  lambda r, sz: acc_out.at[c, pl.ds(r, sz)], base)


@functools.partial(
    pl.kernel,
    out_type=(jax.ShapeDtypeStruct((NC, N_PAD, D), jnp.float32),),
    mesh=_MESH,
    scratch_types=[
        pltpu.VMEM((8, CHUNK), jnp.int32),
        pltpu.VMEM((CHUNK, D), jnp.float32),
        pltpu.VMEM((CHUNK, D), jnp.float32),
        pltpu.VMEM_SHARED((N_PAD, D), jnp.float32),
    ])
def _sc_cnt(dst_hbm, zeros_hbm, ones_hbm, cnt_out,
            dst_v, onesrows, zbuf, acc_sh):
  c = lax.axis_index("c")
  s = lax.axis_index("s")
  wid = s * NC + c
  base = s * RPT

  pltpu.sync_copy(zeros_hbm, zbuf)
  pltpu.sync_copy(ones_hbm, onesrows)
  _acc_slice_copy(lambda r, sz: zbuf.at[pl.ds(0, sz)],
                  lambda r, sz: acc_sh.at[pl.ds(r, sz)], base)
  plsc.subcore_barrier()

  def chunk_step(t, carry):
    pltpu.sync_copy(dst_hbm.at[wid, pl.ds(t * 8, 8)], dst_v)
    for k in range(8):
      pltpu.sync_copy(onesrows, acc_sh.at[dst_v.at[k]], add=True)
    return carry
  lax.fori_loop(0, CHUNKS // 8, chunk_step, 0)

  plsc.subcore_barrier()

  _acc_slice_copy(lambda r, sz: acc_sh.at[pl.ds(r, sz)],
                  lambda r, sz: zbuf.at[pl.ds(0, sz)], base)
  _acc_slice_copy(lambda r, sz: zbuf.at[pl.ds(0, sz)],
                  lambda r, sz: cnt_out.at[c, pl.ds(r, sz)], base)


_BLK = 632


def _dot(a, b):
  return jnp.dot(a, b, preferred_element_type=jnp.float32, precision=_HIGH)


def _proj_body(x_ref, w_ref, b_ref, o_ref):
  o_ref[...] = jnp.maximum(_dot(x_ref[...], w_ref[...]) + b_ref[...], 0.0)


def _proj(x, W, b):
  return pl.pallas_call(
      _proj_body,
      grid=(N_PAD // _BLK,),
      in_specs=[
          pl.BlockSpec((_BLK, D), lambda i: (i, 0)),
          pl.BlockSpec((D, D), lambda i: (0, 0)),
          pl.BlockSpec((D,), lambda i: (0,)),
      ],
      out_specs=pl.BlockSpec((_BLK, D), lambda i: (i, 0)),
      out_shape=jax.ShapeDtypeStruct((N_PAD, D), jnp.float32),
  )(x, W, b)


def _mean_agg(acc_ref, cnt_ref):
  ssum = acc_ref[0] + acc_ref[1]
  cnt = cnt_ref[0][:, 0:1] + cnt_ref[1][:, 0:1]
  return ssum / jnp.clip(cnt, 1.0, None)


def _combine1_body(acc_ref, cnt_ref, x_ref, wl_ref, bl_ref, wr_ref,
                   wp2_ref, bp2_ref, h_ref, xp2_ref):
  agg = _mean_agg(acc_ref, cnt_ref)
  h = jnp.maximum(
      _dot(agg, wl_ref[...]) + bl_ref[...] + _dot(x_ref[...], wr_ref[...]),
      0.0)
  h_ref[...] = h
  xp2_ref[...] = jnp.maximum(_dot(h, wp2_ref[...]) + bp2_ref[...], 0.0)


def _combine1(acc, cnt, x, Wl, bl, Wr, Wp2, bp2):
  return pl.pallas_call(
      _combine1_body,
      grid=(N_PAD // _BLK,),
      in_specs=[
          pl.BlockSpec((NC, _BLK, D), lambda i: (0, i, 0)),
          pl.BlockSpec((NC, _BLK, D), lambda i: (0, i, 0)),
          pl.BlockSpec((_BLK, D), lambda i: (i, 0)),
          pl.BlockSpec((D, D), lambda i: (0, 0)),
          pl.BlockSpec((D,), lambda i: (0,)),
          pl.BlockSpec((D, D), lambda i: (0, 0)),
          pl.BlockSpec((D, D), lambda i: (0, 0)),
          pl.BlockSpec((D,), lambda i: (0,)),
      ],
      out_specs=[
          pl.BlockSpec((_BLK, D), lambda i: (i, 0)),
          pl.BlockSpec((_BLK, D), lambda i: (i, 0)),
      ],
      out_shape=[
          jax.ShapeDtypeStruct((N_PAD, D), jnp.float32),
          jax.ShapeDtypeStruct((N_PAD, D), jnp.float32),
      ],
  )(acc, cnt, x, Wl, bl, Wr, Wp2, bp2)


def _combine2_body(acc_ref, cnt_ref, h_ref, wl_ref, bl_ref, wr_ref, o_ref):
  agg = _mean_agg(acc_ref, cnt_ref)
  o = (_dot(agg, wl_ref[...]) + bl_ref[...] + _dot(h_ref[...], wr_ref[...]))
  norm = jnp.sqrt(jnp.sum(o * o, axis=-1, keepdims=True))
  o = o / jnp.clip(norm, 1e-12, None)
  o = jnp.maximum(o, 0.0)
  m = jnp.max(o, axis=-1, keepdims=True)
  lse = m + jnp.log(jnp.sum(jnp.exp(o - m), axis=-1, keepdims=True))
  o_ref[...] = o - lse


def _combine2(acc, cnt, h, Wl, bl, Wr):
  return pl.pallas_call(
      _combine2_body,
      grid=(N_PAD // _BLK,),
      in_specs=[
          pl.BlockSpec((NC, _BLK, D), lambda i: (0, i, 0)),
          pl.BlockSpec((NC, _BLK, D), lambda i: (0, i, 0)),
          pl.BlockSpec((_BLK, D), lambda i: (i, 0)),
          pl.BlockSpec((D, D), lambda i: (0, 0)),
          pl.BlockSpec((D,), lambda i: (0,)),
          pl.BlockSpec((D, D), lambda i: (0, 0)),
      ],
      out_specs=pl.BlockSpec((_BLK, D), lambda i: (i, 0)),
      out_shape=jax.ShapeDtypeStruct((N_PAD, D), jnp.float32),
  )(acc, cnt, h, Wl, bl, Wr)


def kernel(matrix_nodes_features, edge_index, Wp1, bp1, Wl1, bl1, Wr1,
           Wp2, bp2, Wl2, bl2, Wr2):
  x = jnp.pad(matrix_nodes_features, ((0, N_PAD - N), (0, 0)))
  src = jnp.concatenate(
      [edge_index[0], jnp.zeros((E_PAD - E,), jnp.int32)]).reshape(
          NW, CHUNKS, CHUNK)
  dst = jnp.concatenate(
      [edge_index[1], jnp.full((E_PAD - E,), N, jnp.int32)]).reshape(
          NW, CHUNKS, CHUNK)
  zeros = jnp.zeros((CHUNK, D), jnp.float32)
  ones = jnp.ones((CHUNK, D), jnp.float32)

  (cnt,) = _sc_cnt(dst, zeros, ones)
  xp1 = _proj(x, Wp1, bp1)
  (acc1,) = _sc_agg(xp1, src, dst, zeros)
  h, xp2 = _combine1(acc1, cnt, x, Wl1, bl1, Wr1, Wp2, bp2)
  (acc2,) = _sc_agg(xp2, src, dst, zeros)
  out = _combine2(acc2, cnt, h, Wl2, bl2, Wr2)
  return out[:N]


# speedup vs baseline: 2.8291x; 1.0268x over previous
import functools

import jax
import jax.numpy as jnp
from jax import lax
from jax.experimental import pallas as pl
from jax.experimental.pallas import tpu as pltpu
from jax.experimental.pallas import tpu_sc as plsc

N, E, D = 10000, 320000, 128
NC, NS = 2, 16
NW = NC * NS
CHUNK = 64
CHUNKS = 160
EPW = CHUNK * CHUNKS
E_PAD = EPW * NW
N_PAD = 10112
RPT = N_PAD // NS
_HIGH = lax.Precision.HIGHEST
_MESH = plsc.VectorSubcoreMesh(core_axis_name="c", subcore_axis_name="s")
_SLICES = tuple([CHUNK] * (RPT // CHUNK) +
                ([RPT % CHUNK] if RPT % CHUNK else []))


def _acc_slice_copy(src_at, dst_at, base):
  off = 0
  for sz in _SLICES:
    pltpu.sync_copy(src_at(base + off, sz), dst_at(base + off, sz))
    off += sz


@functools.partial(
    pl.kernel,
    out_type=(jax.ShapeDtypeStruct((NC, N_PAD, D), jnp.float32),),
    mesh=_MESH,
    scratch_types=[
        pltpu.VMEM((8, CHUNK), jnp.int32),
        pltpu.VMEM((8, CHUNK), jnp.int32),
        pltpu.VMEM((CHUNK, D), jnp.float32),
        pltpu.VMEM((CHUNK, D), jnp.float32),
        pltpu.VMEM_SHARED((N_PAD, D), jnp.float32),
        pltpu.SemaphoreType.DMA,
    ])
def _sc_agg(xp_hbm, src_hbm, dst_hbm, zeros_hbm, acc_out,
            src_v, dst_v, rowbuf, rowbuf2, acc_sh, sem):
  c = lax.axis_index("c")
  s = lax.axis_index("s")
  wid = s * NC + c
  base = s * RPT
  rb = (rowbuf, rowbuf2)

  pltpu.sync_copy(zeros_hbm, rowbuf)
  _acc_slice_copy(lambda r, sz: rowbuf.at[pl.ds(0, sz)],
                  lambda r, sz: acc_sh.at[pl.ds(r, sz)], base)
  plsc.subcore_barrier()

  def chunk_step(t, carry):
    pltpu.sync_copy(src_hbm.at[wid, pl.ds(t * 8, 8)], src_v)
    pltpu.sync_copy(dst_hbm.at[wid, pl.ds(t * 8, 8)], dst_v)
    pltpu.async_copy(xp_hbm.at[src_v.at[0]], rb[0], sem)
    for k in range(8):
      if k + 1 < 8:
        pltpu.async_copy(xp_hbm.at[src_v.at[k + 1]], rb[(k + 1) % 2], sem)
      pltpu.make_async_copy(xp_hbm.at[src_v.at[k]], rb[k % 2], sem).wait()
      pltpu.sync_copy(rb[k % 2], acc_sh.at[dst_v.at[k]], add=True)
    return carry
  lax.fori_loop(0, CHUNKS // 8, chunk_step, 0)

  plsc.subcore_barrier()

  _acc_slice_copy(lambda r, sz: acc_sh.at[pl.ds(r, sz)],
                  lambda r, sz: rowbuf.at[pl.ds(0, sz)], base)
  _acc_slice_copy(lambda r, sz: rowbuf.at[pl.ds(0, sz)],
                  lambda r, sz: acc_out.at[c, pl.ds(r, sz)], base)


@functools.partial(
    pl.kernel,
    out_type=(jax.ShapeDtypeStruct((NC, N_PAD, D), jnp.float32),),
    mesh=_MESH,
    scratch_types=[
        pltpu.VMEM((8, CHUNK), jnp.int32),
        pltpu.VMEM((CHUNK, D), jnp.float32),
        pltpu.VMEM((CHUNK, D), jnp.float32),
        pltpu.VMEM_SHARED((N_PAD, D), jnp.float32),
    ])
def _sc_cnt(dst_hbm, zeros_hbm, ones_hbm, cnt_out,
            dst_v, onesrows, zbuf, acc_sh):
  c = lax.axis_index("c")
  s = lax.axis_index("s")
  wid = s * NC + c
  base = s * RPT

  pltpu.sync_copy(zeros_hbm, zbuf)
  pltpu.sync_copy(ones_hbm, onesrows)
  _acc_slice_copy(lambda r, sz: zbuf.at[pl.ds(0, sz)],
                  lambda r, sz: acc_sh.at[pl.ds(r, sz)], base)
  plsc.subcore_barrier()

  def chunk_step(t, carry):
    pltpu.sync_copy(dst_hbm.at[wid, pl.ds(t * 8, 8)], dst_v)
    for k in range(8):
      pltpu.sync_copy(onesrows, acc_sh.at[dst_v.at[k]], add=True)
    return carry
  lax.fori_loop(0, CHUNKS // 8, chunk_step, 0)

  plsc.subcore_barrier()

  _acc_slice_copy(lambda r, sz: acc_sh.at[pl.ds(r, sz)],
                  lambda r, sz: zbuf.at[pl.ds(0, sz)], base)
  _acc_slice_copy(lambda r, sz: zbuf.at[pl.ds(0, sz)],
                  lambda r, sz: cnt_out.at[c, pl.ds(r, sz)], base)


_BLK = 632


def _dot(a, b):
  return jnp.dot(a, b, preferred_element_type=jnp.float32)


def _proj_body(x_ref, w_ref, b_ref, o_ref):
  o_ref[...] = jnp.maximum(_dot(x_ref[...], w_ref[...]) + b_ref[...], 0.0)


def _proj(x, W, b):
  return pl.pallas_call(
      _proj_body,
      grid=(N_PAD // _BLK,),
      in_specs=[
          pl.BlockSpec((_BLK, D), lambda i: (i, 0)),
          pl.BlockSpec((D, D), lambda i: (0, 0)),
          pl.BlockSpec((D,), lambda i: (0,)),
      ],
      out_specs=pl.BlockSpec((_BLK, D), lambda i: (i, 0)),
      out_shape=jax.ShapeDtypeStruct((N_PAD, D), jnp.float32),
  )(x, W, b)


def _mean_agg(acc_ref, cnt_ref):
  ssum = acc_ref[0] + acc_ref[1]
  cnt = cnt_ref[0][:, 0:1] + cnt_ref[1][:, 0:1]
  return ssum / jnp.clip(cnt, 1.0, None)


def _combine1_body(acc_ref, cnt_ref, x_ref, wl_ref, bl_ref, wr_ref,
                   wp2_ref, bp2_ref, h_ref, xp2_ref):
  agg = _mean_agg(acc_ref, cnt_ref)
  h = jnp.maximum(
      _dot(agg, wl_ref[...]) + bl_ref[...] + _dot(x_ref[...], wr_ref[...]),
      0.0)
  h_ref[...] = h
  xp2_ref[...] = jnp.maximum(_dot(h, wp2_ref[...]) + bp2_ref[...], 0.0)


def _combine1(acc, cnt, x, Wl, bl, Wr, Wp2, bp2):
  return pl.pallas_call(
      _combine1_body,
      grid=(N_PAD // _BLK,),
      in_specs=[
          pl.BlockSpec((NC, _BLK, D), lambda i: (0, i, 0)),
          pl.BlockSpec((NC, _BLK, D), lambda i: (0, i, 0)),
          pl.BlockSpec((_BLK, D), lambda i: (i, 0)),
          pl.BlockSpec((D, D), lambda i: (0, 0)),
          pl.BlockSpec((D,), lambda i: (0,)),
          pl.BlockSpec((D, D), lambda i: (0, 0)),
          pl.BlockSpec((D, D), lambda i: (0, 0)),
          pl.BlockSpec((D,), lambda i: (0,)),
      ],
      out_specs=[
          pl.BlockSpec((_BLK, D), lambda i: (i, 0)),
          pl.BlockSpec((_BLK, D), lambda i: (i, 0)),
      ],
      out_shape=[
          jax.ShapeDtypeStruct((N_PAD, D), jnp.float32),
          jax.ShapeDtypeStruct((N_PAD, D), jnp.float32),
      ],
  )(acc, cnt, x, Wl, bl, Wr, Wp2, bp2)


def _combine2_body(acc_ref, cnt_ref, h_ref, wl_ref, bl_ref, wr_ref, o_ref):
  agg = _mean_agg(acc_ref, cnt_ref)
  o = (_dot(agg, wl_ref[...]) + bl_ref[...] + _dot(h_ref[...], wr_ref[...]))
  norm = jnp.sqrt(jnp.sum(o * o, axis=-1, keepdims=True))
  o = o / jnp.clip(norm, 1e-12, None)
  o = jnp.maximum(o, 0.0)
  m = jnp.max(o, axis=-1, keepdims=True)
  lse = m + jnp.log(jnp.sum(jnp.exp(o - m), axis=-1, keepdims=True))
  o_ref[...] = o - lse


def _combine2(acc, cnt, h, Wl, bl, Wr):
  return pl.pallas_call(
      _combine2_body,
      grid=(N_PAD // _BLK,),
      in_specs=[
          pl.BlockSpec((NC, _BLK, D), lambda i: (0, i, 0)),
          pl.BlockSpec((NC, _BLK, D), lambda i: (0, i, 0)),
          pl.BlockSpec((_BLK, D), lambda i: (i, 0)),
          pl.BlockSpec((D, D), lambda i: (0, 0)),
          pl.BlockSpec((D,), lambda i: (0,)),
          pl.BlockSpec((D, D), lambda i: (0, 0)),
      ],
      out_specs=pl.BlockSpec((_BLK, D), lambda i: (i, 0)),
      out_shape=jax.ShapeDtypeStruct((N_PAD, D), jnp.float32),
  )(acc, cnt, h, Wl, bl, Wr)


def kernel(matrix_nodes_features, edge_index, Wp1, bp1, Wl1, bl1, Wr1,
           Wp2, bp2, Wl2, bl2, Wr2):
  x = jnp.pad(matrix_nodes_features, ((0, N_PAD - N), (0, 0)))
  src = jnp.concatenate(
      [edge_index[0], jnp.zeros((E_PAD - E,), jnp.int32)]).reshape(
          NW, CHUNKS, CHUNK)
  dst = jnp.concatenate(
      [edge_index[1], jnp.full((E_PAD - E,), N, jnp.int32)]).reshape(
          NW, CHUNKS, CHUNK)
  zeros = jnp.zeros((CHUNK, D), jnp.float32)
  ones = jnp.ones((CHUNK, D), jnp.float32)

  (cnt,) = _sc_cnt(dst, zeros, ones)
  xp1 = _proj(x, Wp1, bp1)
  (acc1,) = _sc_agg(xp1, src, dst, zeros)
  h, xp2 = _combine1(acc1, cnt, x, Wl1, bl1, Wr1, Wp2, bp2)
  (acc2,) = _sc_agg(xp2, src, dst, zeros)
  out = _combine2(acc2, cnt, h, Wl2, bl2, Wr2)
  return out[:N]


# speedup vs baseline: 7.6774x; 2.7137x over previous
import functools

import jax
import jax.numpy as jnp
from jax import lax
from jax.experimental import pallas as pl
from jax.experimental.pallas import tpu as pltpu
from jax.experimental.pallas import tpu_sc as plsc

N, E, D = 10000, 320000, 128
NC, NS = 2, 16
NW = NC * NS
CHUNK = 64
CHUNKS = 160
EPW = CHUNK * CHUNKS
E_PAD = EPW * NW
N_PAD = 10112
RPT = N_PAD // NS
_HIGH = lax.Precision.HIGHEST
_MESH = plsc.VectorSubcoreMesh(core_axis_name="c", subcore_axis_name="s")
_SLICES = tuple([CHUNK] * (RPT // CHUNK) +
                ([RPT % CHUNK] if RPT % CHUNK else []))


def _acc_slice_copy(src_at, dst_at, base):
  off = 0
  for sz in _SLICES:
    pltpu.sync_copy(src_at(base + off, sz), dst_at(base + off, sz))
    off += sz


@functools.partial(
    pl.kernel,
    out_type=(jax.ShapeDtypeStruct((NC, N_PAD, D), jnp.float32),),
    mesh=_MESH,
    scratch_types=[
        pltpu.VMEM((8, CHUNK), jnp.int32),
        pltpu.VMEM((8, CHUNK), jnp.int32),
        pltpu.VMEM((CHUNK, D), jnp.float32),
        pltpu.VMEM((CHUNK, D), jnp.float32),
        pltpu.VMEM_SHARED((N_PAD, D), jnp.float32),
        pltpu.SemaphoreType.DMA,
    ])
def _sc_agg(xp_hbm, src_hbm, dst_hbm, zeros_hbm, acc_out,
            src_v, dst_v, rowbuf, rowbuf2, acc_sh, sem):
  c = lax.axis_index("c")
  s = lax.axis_index("s")
  wid = s * NC + c
  base = s * RPT
  rb = (rowbuf, rowbuf2)

  pltpu.sync_copy(zeros_hbm, rowbuf)
  _acc_slice_copy(lambda r, sz: rowbuf.at[pl.ds(0, sz)],
                  lambda r, sz: acc_sh.at[pl.ds(r, sz)], base)
  plsc.subcore_barrier()

  def chunk_step(t, carry):
    pltpu.sync_copy(src_hbm.at[wid, pl.ds(t * 8, 8)], src_v)
    pltpu.sync_copy(dst_hbm.at[wid, pl.ds(t * 8, 8)], dst_v)
    pltpu.async_copy(xp_hbm.at[src_v.at[0]], rb[0], sem)
    for k in range(8):
      if k + 1 < 8:
        pltpu.async_copy(xp_hbm.at[src_v.at[k + 1]], rb[(k + 1) % 2], sem)
      pltpu.make_async_copy(xp_hbm.at[src_v.at[k]], rb[k % 2], sem).wait()
      pltpu.sync_copy(rb[k % 2], acc_sh.at[dst_v.at[k]], add=True)
    return carry
  lax.fori_loop(0, CHUNKS // 8, chunk_step, 0)

  plsc.subcore_barrier()

  _acc_slice_copy(lambda r, sz: acc_sh.at[pl.ds(r, sz)],
                  lambda r, sz: rowbuf.at[pl.ds(0, sz)], base)
  _acc_slice_copy(lambda r, sz: rowbuf.at[pl.ds(0, sz)],
                  lambda r, sz: acc_out.at[c, pl.ds(r, sz)], base)


@functools.partial(
    pl.kernel,
    out_type=(jax.ShapeDtypeStruct((NC, N_PAD, D), jnp.float32),),
    mesh=_MESH,
    scratch_types=[
        pltpu.VMEM((8, CHUNK), jnp.int32),
        pltpu.VMEM((CHUNK, D), jnp.float32),
        pltpu.VMEM((CHUNK, D), jnp.float32),
        pltpu.VMEM_SHARED((N_PAD, D), jnp.float32),
    ])
def _sc_cnt(dst_hbm, zeros_hbm, ones_hbm, cnt_out,
            dst_v, onesrows, zbuf, acc_sh):
  c = lax.axis_index("c")
  s = lax.axis_index("s")
  wid = s * NC + c
  base = s * RPT

  pltpu.sync_copy(zeros_hbm, zbuf)
  pltpu.sync_copy(ones_hbm, onesrows)
  _acc_slice_copy(lambda r, sz: zbuf.at[pl.ds(0, sz)],
                  lambda r, sz: acc_sh.at[pl.ds(r, sz)], base)
  plsc.subcore_barrier()

  def chunk_step(t, carry):
    pltpu.sync_copy(dst_hbm.at[wid, pl.ds(t * 8, 8)], dst_v)
    for k in range(8):
      pltpu.sync_copy(onesrows, acc_sh.at[dst_v.at[k]], add=True)
    return carry
  lax.fori_loop(0, CHUNKS // 8, chunk_step, 0)

  plsc.subcore_barrier()

  _acc_slice_copy(lambda r, sz: acc_sh.at[pl.ds(r, sz)],
                  lambda r, sz: zbuf.at[pl.ds(0, sz)], base)
  _acc_slice_copy(lambda r, sz: zbuf.at[pl.ds(0, sz)],
                  lambda r, sz: cnt_out.at[c, pl.ds(r, sz)], base)


_BLK = 632


def _dot(a, b):
  return jnp.dot(a, b, preferred_element_type=jnp.float32)


def _proj_body(x_ref, w_ref, b_ref, o_ref):
  o_ref[...] = jnp.maximum(_dot(x_ref[...], w_ref[...]) + b_ref[...], 0.0)


def _proj(x, W, b):
  return pl.pallas_call(
      _proj_body,
      grid=(N_PAD // _BLK,),
      in_specs=[
          pl.BlockSpec((_BLK, D), lambda i: (i, 0)),
          pl.BlockSpec((D, D), lambda i: (0, 0)),
          pl.BlockSpec((D,), lambda i: (0,)),
      ],
      out_specs=pl.BlockSpec((_BLK, D), lambda i: (i, 0)),
      out_shape=jax.ShapeDtypeStruct((N_PAD, D), jnp.float32),
  )(x, W, b)


def _mean_agg(acc_ref, cnt_ref):
  ssum = acc_ref[0] + acc_ref[1]
  cnt = cnt_ref[0][:, 0:1] + cnt_ref[1][:, 0:1]
  return ssum / jnp.clip(cnt, 1.0, None)


def _combine1_body(acc_ref, cnt_ref, x_ref, wl_ref, bl_ref, wr_ref,
                   wp2_ref, bp2_ref, h_ref, xp2_ref):
  agg = _mean_agg(acc_ref, cnt_ref)
  h = jnp.maximum(
      _dot(agg, wl_ref[...]) + bl_ref[...] + _dot(x_ref[...], wr_ref[...]),
      0.0)
  h_ref[...] = h
  xp2_ref[...] = jnp.maximum(_dot(h, wp2_ref[...]) + bp2_ref[...], 0.0)


def _combine1(acc, cnt, x, Wl, bl, Wr, Wp2, bp2):
  return pl.pallas_call(
      _combine1_body,
      grid=(N_PAD // _BLK,),
      in_specs=[
          pl.BlockSpec((NC, _BLK, D), lambda i: (0, i, 0)),
          pl.BlockSpec((NC, _BLK, D), lambda i: (0, i, 0)),
          pl.BlockSpec((_BLK, D), lambda i: (i, 0)),
          pl.BlockSpec((D, D), lambda i: (0, 0)),
          pl.BlockSpec((D,), lambda i: (0,)),
          pl.BlockSpec((D, D), lambda i: (0, 0)),
          pl.BlockSpec((D, D), lambda i: (0, 0)),
          pl.BlockSpec((D,), lambda i: (0,)),
      ],
      out_specs=[
          pl.BlockSpec((_BLK, D), lambda i: (i, 0)),
          pl.BlockSpec((_BLK, D), lambda i: (i, 0)),
      ],
      out_shape=[
          jax.ShapeDtypeStruct((N_PAD, D), jnp.float32),
          jax.ShapeDtypeStruct((N_PAD, D), jnp.float32),
      ],
  )(acc, cnt, x, Wl, bl, Wr, Wp2, bp2)


def _combine2_body(acc_ref, cnt_ref, h_ref, wl_ref, bl_ref, wr_ref, o_ref):
  agg = _mean_agg(acc_ref, cnt_ref)
  o = (_dot(agg, wl_ref[...]) + bl_ref[...] + _dot(h_ref[...], wr_ref[...]))
  norm = jnp.sqrt(jnp.sum(o * o, axis=-1, keepdims=True))
  o = o / jnp.clip(norm, 1e-12, None)
  o = jnp.maximum(o, 0.0)
  m = jnp.max(o, axis=-1, keepdims=True)
  lse = m + jnp.log(jnp.sum(jnp.exp(o - m), axis=-1, keepdims=True))
  o_ref[...] = o - lse


def _combine2(acc, cnt, h, Wl, bl, Wr):
  return pl.pallas_call(
      _combine2_body,
      grid=(N_PAD // _BLK,),
      in_specs=[
          pl.BlockSpec((NC, _BLK, D), lambda i: (0, i, 0)),
          pl.BlockSpec((NC, _BLK, D), lambda i: (0, i, 0)),
          pl.BlockSpec((_BLK, D), lambda i: (i, 0)),
          pl.BlockSpec((D, D), lambda i: (0, 0)),
          pl.BlockSpec((D,), lambda i: (0,)),
          pl.BlockSpec((D, D), lambda i: (0, 0)),
      ],
      out_specs=pl.BlockSpec((_BLK, D), lambda i: (i, 0)),
      out_shape=jax.ShapeDtypeStruct((N_PAD, D), jnp.float32),
  )(acc, cnt, h, Wl, bl, Wr)


def kernel(matrix_nodes_features, edge_index, Wp1, bp1, Wl1, bl1, Wr1,
           Wp2, bp2, Wl2, bl2, Wr2):
  x = jnp.pad(matrix_nodes_features, ((0, N_PAD - N), (0, 0)))
  pad_i = jnp.arange(E_PAD - E, dtype=jnp.int32)
  src = jnp.concatenate(
      [edge_index[0], (pad_i * 131) % N]).reshape(NW, CHUNKS, CHUNK)
  dst = jnp.concatenate(
      [edge_index[1], N + pad_i % (N_PAD - N)]).reshape(NW, CHUNKS, CHUNK)
  zeros = jnp.zeros((CHUNK, D), jnp.float32)
  ones = jnp.ones((CHUNK, D), jnp.float32)

  (cnt,) = _sc_cnt(dst, zeros, ones)
  xp1 = _proj(x, Wp1, bp1)
  (acc1,) = _sc_agg(xp1, src, dst, zeros)
  h, xp2 = _combine1(acc1, cnt, x, Wl1, bl1, Wr1, Wp2, bp2)
  (acc2,) = _sc_agg(xp2, src, dst, zeros)
  out = _combine2(acc2, cnt, h, Wl2, bl2, Wr2)
  return out[:N]


# speedup vs baseline: 8.1889x; 1.0666x over previous
import functools

import jax
import jax.numpy as jnp
from jax import lax
from jax.experimental import pallas as pl
from jax.experimental.pallas import tpu as pltpu
from jax.experimental.pallas import tpu_sc as plsc

N, E, D = 10000, 320000, 128
NC, NS = 2, 16
NW = NC * NS
CHUNK = 80
CHUNKS = 128
EPW = CHUNK * CHUNKS
E_PAD = EPW * NW
N_PAD = 10112
RPT = N_PAD // NS
_HIGH = lax.Precision.HIGHEST
_MESH = plsc.VectorSubcoreMesh(core_axis_name="c", subcore_axis_name="s")
_SLICES = tuple([CHUNK] * (RPT // CHUNK) +
                ([RPT % CHUNK] if RPT % CHUNK else []))


def _acc_slice_copy(src_at, dst_at, base):
  off = 0
  for sz in _SLICES:
    pltpu.sync_copy(src_at(base + off, sz), dst_at(base + off, sz))
    off += sz


@functools.partial(
    pl.kernel,
    out_type=(jax.ShapeDtypeStruct((NC, N_PAD, D), jnp.float32),),
    mesh=_MESH,
    scratch_types=[
        pltpu.VMEM((8, CHUNK), jnp.int32),
        pltpu.VMEM((8, CHUNK), jnp.int32),
        pltpu.VMEM((CHUNK, D), jnp.float32),
        pltpu.VMEM((CHUNK, D), jnp.float32),
        pltpu.VMEM_SHARED((N_PAD, D), jnp.float32),
        pltpu.SemaphoreType.DMA,
    ])
def _sc_agg(xp_hbm, src_hbm, dst_hbm, zeros_hbm, acc_out,
            src_v, dst_v, rowbuf, rowbuf2, acc_sh, sem):
  c = lax.axis_index("c")
  s = lax.axis_index("s")
  wid = s * NC + c
  base = s * RPT
  rb = (rowbuf, rowbuf2)

  pltpu.sync_copy(zeros_hbm, rowbuf)
  _acc_slice_copy(lambda r, sz: rowbuf.at[pl.ds(0, sz)],
                  lambda r, sz: acc_sh.at[pl.ds(r, sz)], base)
  plsc.subcore_barrier()

  def chunk_step(t, carry):
    pltpu.sync_copy(src_hbm.at[wid, pl.ds(t * 8, 8)], src_v)
    pltpu.sync_copy(dst_hbm.at[wid, pl.ds(t * 8, 8)], dst_v)
    pltpu.async_copy(xp_hbm.at[src_v.at[0]], rb[0], sem)
    for k in range(8):
      if k + 1 < 8:
        pltpu.async_copy(xp_hbm.at[src_v.at[k + 1]], rb[(k + 1) % 2], sem)
      pltpu.make_async_copy(xp_hbm.at[src_v.at[k]], rb[k % 2], sem).wait()
      pltpu.sync_copy(rb[k % 2], acc_sh.at[dst_v.at[k]], add=True)
    return carry
  lax.fori_loop(0, CHUNKS // 8, chunk_step, 0)

  plsc.subcore_barrier()

  _acc_slice_copy(lambda r, sz: acc_sh.at[pl.ds(r, sz)],
                  lambda r, sz: rowbuf.at[pl.ds(0, sz)], base)
  _acc_slice_copy(lambda r, sz: rowbuf.at[pl.ds(0, sz)],
                  lambda r, sz: acc_out.at[c, pl.ds(r, sz)], base)


@functools.partial(
    pl.kernel,
    out_type=(jax.ShapeDtypeStruct((NC, N_PAD, D), jnp.float32),),
    mesh=_MESH,
    scratch_types=[
        pltpu.VMEM((8, CHUNK), jnp.int32),
        pltpu.VMEM((CHUNK, D), jnp.float32),
        pltpu.VMEM((CHUNK, D), jnp.float32),
        pltpu.VMEM_SHARED((N_PAD, D), jnp.float32),
    ])
def _sc_cnt(dst_hbm, zeros_hbm, ones_hbm, cnt_out,
            dst_v, onesrows, zbuf, acc_sh):
  c = lax.axis_index("c")
  s = lax.axis_index("s")
  wid = s * NC + c
  base = s * RPT

  pltpu.sync_copy(zeros_hbm, zbuf)
  pltpu.sync_copy(ones_hbm, onesrows)
  _acc_slice_copy(lambda r, sz: zbuf.at[pl.ds(0, sz)],
                  lambda r, sz: acc_sh.at[pl.ds(r, sz)], base)
  plsc.subcore_barrier()

  def chunk_step(t, carry):
    pltpu.sync_copy(dst_hbm.at[wid, pl.ds(t * 8, 8)], dst_v)
    for k in range(8):
      pltpu.sync_copy(onesrows, acc_sh.at[dst_v.at[k]], add=True)
    return carry
  lax.fori_loop(0, CHUNKS // 8, chunk_step, 0)

  plsc.subcore_barrier()

  _acc_slice_copy(lambda r, sz: acc_sh.at[pl.ds(r, sz)],
                  lambda r, sz: zbuf.at[pl.ds(0, sz)], base)
  _acc_slice_copy(lambda r, sz: zbuf.at[pl.ds(0, sz)],
                  lambda r, sz: cnt_out.at[c, pl.ds(r, sz)], base)


_BLK = 632


def _dot(a, b):
  return jnp.dot(a, b, preferred_element_type=jnp.float32)


def _proj_body(x_ref, w_ref, b_ref, o_ref):
  o_ref[...] = jnp.maximum(_dot(x_ref[...], w_ref[...]) + b_ref[...], 0.0)


def _proj(x, W, b):
  return pl.pallas_call(
      _proj_body,
      grid=(N_PAD // _BLK,),
      in_specs=[
          pl.BlockSpec((_BLK, D), lambda i: (i, 0)),
          pl.BlockSpec((D, D), lambda i: (0, 0)),
          pl.BlockSpec((D,), lambda i: (0,)),
      ],
      out_specs=pl.BlockSpec((_BLK, D), lambda i: (i, 0)),
      out_shape=jax.ShapeDtypeStruct((N_PAD, D), jnp.float32),
  )(x, W, b)


def _mean_agg(acc_ref, cnt_ref):
  ssum = acc_ref[0] + acc_ref[1]
  cnt = cnt_ref[0][:, 0:1] + cnt_ref[1][:, 0:1]
  return ssum / jnp.clip(cnt, 1.0, None)


def _combine1_body(acc_ref, cnt_ref, x_ref, wl_ref, bl_ref, wr_ref,
                   wp2_ref, bp2_ref, h_ref, xp2_ref):
  agg = _mean_agg(acc_ref, cnt_ref)
  h = jnp.maximum(
      _dot(agg, wl_ref[...]) + bl_ref[...] + _dot(x_ref[...], wr_ref[...]),
      0.0)
  h_ref[...] = h
  xp2_ref[...] = jnp.maximum(_dot(h, wp2_ref[...]) + bp2_ref[...], 0.0)


def _combine1(acc, cnt, x, Wl, bl, Wr, Wp2, bp2):
  return pl.pallas_call(
      _combine1_body,
      grid=(N_PAD // _BLK,),
      in_specs=[
          pl.BlockSpec((NC, _BLK, D), lambda i: (0, i, 0)),
          pl.BlockSpec((NC, _BLK, D), lambda i: (0, i, 0)),
          pl.BlockSpec((_BLK, D), lambda i: (i, 0)),
          pl.BlockSpec((D, D), lambda i: (0, 0)),
          pl.BlockSpec((D,), lambda i: (0,)),
          pl.BlockSpec((D, D), lambda i: (0, 0)),
          pl.BlockSpec((D, D), lambda i: (0, 0)),
          pl.BlockSpec((D,), lambda i: (0,)),
      ],
      out_specs=[
          pl.BlockSpec((_BLK, D), lambda i: (i, 0)),
          pl.BlockSpec((_BLK, D), lambda i: (i, 0)),
      ],
      out_shape=[
          jax.ShapeDtypeStruct((N_PAD, D), jnp.float32),
          jax.ShapeDtypeStruct((N_PAD, D), jnp.float32),
      ],
  )(acc, cnt, x, Wl, bl, Wr, Wp2, bp2)


def _combine2_body(acc_ref, cnt_ref, h_ref, wl_ref, bl_ref, wr_ref, o_ref):
  agg = _mean_agg(acc_ref, cnt_ref)
  o = (_dot(agg, wl_ref[...]) + bl_ref[...] + _dot(h_ref[...], wr_ref[...]))
  norm = jnp.sqrt(jnp.sum(o * o, axis=-1, keepdims=True))
  o = o / jnp.clip(norm, 1e-12, None)
  o = jnp.maximum(o, 0.0)
  m = jnp.max(o, axis=-1, keepdims=True)
  lse = m + jnp.log(jnp.sum(jnp.exp(o - m), axis=-1, keepdims=True))
  o_ref[...] = o - lse


def _combine2(acc, cnt, h, Wl, bl, Wr):
  return pl.pallas_call(
      _combine2_body,
      grid=(N_PAD // _BLK,),
      in_specs=[
          pl.BlockSpec((NC, _BLK, D), lambda i: (0, i, 0)),
          pl.BlockSpec((NC, _BLK, D), lambda i: (0, i, 0)),
          pl.BlockSpec((_BLK, D), lambda i: (i, 0)),
          pl.BlockSpec((D, D), lambda i: (0, 0)),
          pl.BlockSpec((D,), lambda i: (0,)),
          pl.BlockSpec((D, D), lambda i: (0, 0)),
      ],
      out_specs=pl.BlockSpec((_BLK, D), lambda i: (i, 0)),
      out_shape=jax.ShapeDtypeStruct((N_PAD, D), jnp.float32),
  )(acc, cnt, h, Wl, bl, Wr)


def kernel(matrix_nodes_features, edge_index, Wp1, bp1, Wl1, bl1, Wr1,
           Wp2, bp2, Wl2, bl2, Wr2):
  x = jnp.pad(matrix_nodes_features, ((0, N_PAD - N), (0, 0)))
  pad_i = jnp.arange(E_PAD - E, dtype=jnp.int32)
  src = jnp.concatenate(
      [edge_index[0], (pad_i * 131) % N]).reshape(NW, CHUNKS, CHUNK)
  dst = jnp.concatenate(
      [edge_index[1], N + pad_i % (N_PAD - N)]).reshape(NW, CHUNKS, CHUNK)
  zeros = jnp.zeros((CHUNK, D), jnp.float32)
  ones = jnp.ones((CHUNK, D), jnp.float32)

  (cnt,) = _sc_cnt(dst, zeros, ones)
  xp1 = _proj(x, Wp1, bp1)
  (acc1,) = _sc_agg(xp1, src, dst, zeros)
  h, xp2 = _combine1(acc1, cnt, x, Wl1, bl1, Wr1, Wp2, bp2)
  (acc2,) = _sc_agg(xp2, src, dst, zeros)
  out = _combine2(acc2, cnt, h, Wl2, bl2, Wr2)
  return out[:N]


# speedup vs baseline: 8.2312x; 1.0052x over previous
import functools

import numpy as np

import jax
import jax.numpy as jnp
from jax import lax
from jax.experimental import pallas as pl
from jax.experimental.pallas import tpu as pltpu
from jax.experimental.pallas import tpu_sc as plsc

N, E, D = 10000, 320000, 128
NC, NS = 2, 16
NW = NC * NS
CHUNK = 80
CHUNKS = 128
EPW = CHUNK * CHUNKS
E_PAD = EPW * NW
N_PAD = 10112
RPT = N_PAD // NS
_HIGH = lax.Precision.HIGHEST
_MESH = plsc.VectorSubcoreMesh(core_axis_name="c", subcore_axis_name="s")
_SLICES = tuple([CHUNK] * (RPT // CHUNK) +
                ([RPT % CHUNK] if RPT % CHUNK else []))


def _acc_slice_copy(src_at, dst_at, base):
  off = 0
  for sz in _SLICES:
    pltpu.sync_copy(src_at(base + off, sz), dst_at(base + off, sz))
    off += sz


@functools.partial(
    pl.kernel,
    out_type=(jax.ShapeDtypeStruct((NC, N_PAD, D), jnp.float32),),
    mesh=_MESH,
    scratch_types=[
        pltpu.VMEM((8, CHUNK), jnp.int32),
        pltpu.VMEM((8, CHUNK), jnp.int32),
        pltpu.VMEM((CHUNK, D), jnp.float32),
        pltpu.VMEM((CHUNK, D), jnp.float32),
        pltpu.VMEM_SHARED((N_PAD, D), jnp.float32),
        pltpu.SemaphoreType.DMA,
    ])
def _sc_agg(xp_hbm, src_hbm, dst_hbm, zeros_hbm, acc_out,
            src_v, dst_v, rowbuf, rowbuf2, acc_sh, sem):
  c = lax.axis_index("c")
  s = lax.axis_index("s")
  wid = s * NC + c
  base = s * RPT
  rb = (rowbuf, rowbuf2)

  pltpu.sync_copy(zeros_hbm, rowbuf)
  _acc_slice_copy(lambda r, sz: rowbuf.at[pl.ds(0, sz)],
                  lambda r, sz: acc_sh.at[pl.ds(r, sz)], base)
  plsc.subcore_barrier()

  def chunk_step(t, carry):
    pltpu.sync_copy(src_hbm.at[wid, pl.ds(t * 8, 8)], src_v)
    pltpu.sync_copy(dst_hbm.at[wid, pl.ds(t * 8, 8)], dst_v)
    pltpu.async_copy(xp_hbm.at[src_v.at[0]], rb[0], sem)
    for k in range(8):
      if k + 1 < 8:
        pltpu.async_copy(xp_hbm.at[src_v.at[k + 1]], rb[(k + 1) % 2], sem)
      pltpu.make_async_copy(xp_hbm.at[src_v.at[k]], rb[k % 2], sem).wait()
      pltpu.sync_copy(rb[k % 2], acc_sh.at[dst_v.at[k]], add=True)
    return carry
  lax.fori_loop(0, CHUNKS // 8, chunk_step, 0)

  plsc.subcore_barrier()

  _acc_slice_copy(lambda r, sz: acc_sh.at[pl.ds(r, sz)],
                  lambda r, sz: rowbuf.at[pl.ds(0, sz)], base)
  _acc_slice_copy(lambda r, sz: rowbuf.at[pl.ds(0, sz)],
                  lambda r, sz: acc_out.at[c, pl.ds(r, sz)], base)


@functools.partial(
    pl.kernel,
    out_type=(jax.ShapeDtypeStruct((NC, N_PAD, D), jnp.float32),),
    mesh=_MESH,
    scratch_types=[
        pltpu.VMEM((8, CHUNK), jnp.int32),
        pltpu.VMEM((CHUNK, D), jnp.float32),
        pltpu.VMEM((CHUNK, D), jnp.float32),
        pltpu.VMEM_SHARED((N_PAD, D), jnp.float32),
    ])
def _sc_cnt(dst_hbm, zeros_hbm, ones_hbm, cnt_out,
            dst_v, onesrows, zbuf, acc_sh):
  c = lax.axis_index("c")
  s = lax.axis_index("s")
  wid = s * NC + c
  base = s * RPT

  pltpu.sync_copy(zeros_hbm, zbuf)
  pltpu.sync_copy(ones_hbm, onesrows)
  _acc_slice_copy(lambda r, sz: zbuf.at[pl.ds(0, sz)],
                  lambda r, sz: acc_sh.at[pl.ds(r, sz)], base)
  plsc.subcore_barrier()

  def chunk_step(t, carry):
    pltpu.sync_copy(dst_hbm.at[wid, pl.ds(t * 8, 8)], dst_v)
    for k in range(8):
      pltpu.sync_copy(onesrows, acc_sh.at[dst_v.at[k]], add=True)
    return carry
  lax.fori_loop(0, CHUNKS // 8, chunk_step, 0)

  plsc.subcore_barrier()

  _acc_slice_copy(lambda r, sz: acc_sh.at[pl.ds(r, sz)],
                  lambda r, sz: zbuf.at[pl.ds(0, sz)], base)
  _acc_slice_copy(lambda r, sz: zbuf.at[pl.ds(0, sz)],
                  lambda r, sz: cnt_out.at[c, pl.ds(r, sz)], base)


_BLK = 632


def _dot(a, b):
  return jnp.dot(a, b, preferred_element_type=jnp.float32)


def _proj_body(x_ref, w_ref, b_ref, o_ref):
  o_ref[...] = jnp.maximum(_dot(x_ref[...], w_ref[...]) + b_ref[...], 0.0)


def _proj(x, W, b):
  return pl.pallas_call(
      _proj_body,
      grid=(N_PAD // _BLK,),
      in_specs=[
          pl.BlockSpec((_BLK, D), lambda i: (i, 0)),
          pl.BlockSpec((D, D), lambda i: (0, 0)),
          pl.BlockSpec((D,), lambda i: (0,)),
      ],
      out_specs=pl.BlockSpec((_BLK, D), lambda i: (i, 0)),
      out_shape=jax.ShapeDtypeStruct((N_PAD, D), jnp.float32),
  )(x, W, b)


def _mean_agg(acc_ref, cnt_ref):
  ssum = acc_ref[0] + acc_ref[1]
  cnt = cnt_ref[0][:, 0:1] + cnt_ref[1][:, 0:1]
  return ssum / jnp.clip(cnt, 1.0, None)


def _combine1_body(acc_ref, cnt_ref, x_ref, wl_ref, bl_ref, wr_ref,
                   wp2_ref, bp2_ref, h_ref, xp2_ref):
  agg = _mean_agg(acc_ref, cnt_ref)
  h = jnp.maximum(
      _dot(agg, wl_ref[...]) + bl_ref[...] + _dot(x_ref[...], wr_ref[...]),
      0.0)
  h_ref[...] = h
  xp2_ref[...] = jnp.maximum(_dot(h, wp2_ref[...]) + bp2_ref[...], 0.0)


def _combine1(acc, cnt, x, Wl, bl, Wr, Wp2, bp2):
  return pl.pallas_call(
      _combine1_body,
      grid=(N_PAD // _BLK,),
      in_specs=[
          pl.BlockSpec((NC, _BLK, D), lambda i: (0, i, 0)),
          pl.BlockSpec((NC, _BLK, D), lambda i: (0, i, 0)),
          pl.BlockSpec((_BLK, D), lambda i: (i, 0)),
          pl.BlockSpec((D, D), lambda i: (0, 0)),
          pl.BlockSpec((D,), lambda i: (0,)),
          pl.BlockSpec((D, D), lambda i: (0, 0)),
          pl.BlockSpec((D, D), lambda i: (0, 0)),
          pl.BlockSpec((D,), lambda i: (0,)),
      ],
      out_specs=[
          pl.BlockSpec((_BLK, D), lambda i: (i, 0)),
          pl.BlockSpec((_BLK, D), lambda i: (i, 0)),
      ],
      out_shape=[
          jax.ShapeDtypeStruct((N_PAD, D), jnp.float32),
          jax.ShapeDtypeStruct((N_PAD, D), jnp.float32),
      ],
  )(acc, cnt, x, Wl, bl, Wr, Wp2, bp2)


def _combine2_body(acc_ref, cnt_ref, h_ref, wl_ref, bl_ref, wr_ref, o_ref):
  agg = _mean_agg(acc_ref, cnt_ref)
  o = (_dot(agg, wl_ref[...]) + bl_ref[...] + _dot(h_ref[...], wr_ref[...]))
  norm = jnp.sqrt(jnp.sum(o * o, axis=-1, keepdims=True))
  o = o / jnp.clip(norm, 1e-12, None)
  o = jnp.maximum(o, 0.0)
  m = jnp.max(o, axis=-1, keepdims=True)
  lse = m + jnp.log(jnp.sum(jnp.exp(o - m), axis=-1, keepdims=True))
  o_ref[...] = o - lse


def _combine2(acc, cnt, h, Wl, bl, Wr):
  return pl.pallas_call(
      _combine2_body,
      grid=(N_PAD // _BLK,),
      in_specs=[
          pl.BlockSpec((NC, _BLK, D), lambda i: (0, i, 0)),
          pl.BlockSpec((NC, _BLK, D), lambda i: (0, i, 0)),
          pl.BlockSpec((_BLK, D), lambda i: (i, 0)),
          pl.BlockSpec((D, D), lambda i: (0, 0)),
          pl.BlockSpec((D,), lambda i: (0,)),
          pl.BlockSpec((D, D), lambda i: (0, 0)),
      ],
      out_specs=pl.BlockSpec((_BLK, D), lambda i: (i, 0)),
      out_shape=jax.ShapeDtypeStruct((N_PAD, D), jnp.float32),
  )(acc, cnt, h, Wl, bl, Wr)


def kernel(matrix_nodes_features, edge_index, Wp1, bp1, Wl1, bl1, Wr1,
           Wp2, bp2, Wl2, bl2, Wr2):
  x = jnp.pad(matrix_nodes_features, ((0, N_PAD - N), (0, 0)))
  pad_i = np.arange(E_PAD - E, dtype=np.int32)
  pad_src = jnp.asarray((pad_i * 131) % N)
  pad_dst = jnp.asarray(N + pad_i % (N_PAD - N))
  src = jnp.concatenate([edge_index[0], pad_src]).reshape(NW, CHUNKS, CHUNK)
  dst = jnp.concatenate([edge_index[1], pad_dst]).reshape(NW, CHUNKS, CHUNK)
  zeros = jnp.zeros((CHUNK, D), jnp.float32)
  ones = jnp.ones((CHUNK, D), jnp.float32)

  (cnt,) = _sc_cnt(dst, zeros, ones)
  xp1 = _proj(x, Wp1, bp1)
  (acc1,) = _sc_agg(xp1, src, dst, zeros)
  h, xp2 = _combine1(acc1, cnt, x, Wl1, bl1, Wr1, Wp2, bp2)
  (acc2,) = _sc_agg(xp2, src, dst, zeros)
  out = _combine2(acc2, cnt, h, Wl2, bl2, Wr2)
  return out[:N]


# speedup vs baseline: 8.2427x; 1.0014x over previous
import functools

import numpy as np

import jax
import jax.numpy as jnp
from jax import lax
from jax.experimental import pallas as pl
from jax.experimental.pallas import tpu as pltpu
from jax.experimental.pallas import tpu_sc as plsc

N, E, D = 10000, 320000, 128
NC, NS = 2, 16
NW = NC * NS
CHUNK = 80
CHUNKS = 128
EPW = CHUNK * CHUNKS
E_PAD = EPW * NW
N_PAD = 10112
RPT = N_PAD // NS
_HIGH = lax.Precision.HIGHEST
_MESH = plsc.VectorSubcoreMesh(core_axis_name="c", subcore_axis_name="s")
_SLICES = tuple([CHUNK] * (RPT // CHUNK) +
                ([RPT % CHUNK] if RPT % CHUNK else []))


def _acc_slice_copy(src_at, dst_at, base):
  off = 0
  for sz in _SLICES:
    pltpu.sync_copy(src_at(base + off, sz), dst_at(base + off, sz))
    off += sz


@functools.partial(
    pl.kernel,
    out_type=(jax.ShapeDtypeStruct((NC, N_PAD, D), jnp.float32),),
    mesh=_MESH,
    scratch_types=[
        pltpu.VMEM((8, CHUNK), jnp.int32),
        pltpu.VMEM((8, CHUNK), jnp.int32),
        pltpu.VMEM((CHUNK, D), jnp.float32),
        pltpu.VMEM((CHUNK, D), jnp.float32),
        pltpu.VMEM_SHARED((N_PAD, D), jnp.float32),
        pltpu.SemaphoreType.DMA,
        pltpu.SemaphoreType.DMA,
    ])
def _sc_agg(xp_hbm, src_hbm, dst_hbm, zeros_hbm, acc_out,
            src_v, dst_v, rowbuf, rowbuf2, acc_sh, sem, sem2):
  c = lax.axis_index("c")
  s = lax.axis_index("s")
  wid = s * NC + c
  base = s * RPT
  rb = (rowbuf, rowbuf2)

  pltpu.sync_copy(zeros_hbm, rowbuf)
  _acc_slice_copy(lambda r, sz: rowbuf.at[pl.ds(0, sz)],
                  lambda r, sz: acc_sh.at[pl.ds(r, sz)], base)
  plsc.subcore_barrier()

  def chunk_step(t, carry):
    pltpu.sync_copy(src_hbm.at[wid, pl.ds(t * 8, 8)], src_v)
    pltpu.sync_copy(dst_hbm.at[wid, pl.ds(t * 8, 8)], dst_v)
    pltpu.async_copy(xp_hbm.at[src_v.at[0]], rb[0], sem)
    for k in range(8):
      if k + 1 < 8:
        if k >= 1:
          pltpu.make_async_copy(rb[(k - 1) % 2], acc_sh.at[dst_v.at[k - 1]],
                                sem2).wait()
        pltpu.async_copy(xp_hbm.at[src_v.at[k + 1]], rb[(k + 1) % 2], sem)
      pltpu.make_async_copy(xp_hbm.at[src_v.at[k]], rb[k % 2], sem).wait()
      pltpu.async_copy(rb[k % 2], acc_sh.at[dst_v.at[k]], sem2, add=True)
    pltpu.make_async_copy(rb[0], acc_sh.at[dst_v.at[6]], sem2).wait()
    pltpu.make_async_copy(rb[1], acc_sh.at[dst_v.at[7]], sem2).wait()
    return carry
  lax.fori_loop(0, CHUNKS // 8, chunk_step, 0)

  plsc.subcore_barrier()

  _acc_slice_copy(lambda r, sz: acc_sh.at[pl.ds(r, sz)],
                  lambda r, sz: rowbuf.at[pl.ds(0, sz)], base)
  _acc_slice_copy(lambda r, sz: rowbuf.at[pl.ds(0, sz)],
                  lambda r, sz: acc_out.at[c, pl.ds(r, sz)], base)


@functools.partial(
    pl.kernel,
    out_type=(jax.ShapeDtypeStruct((NC, N_PAD, D), jnp.float32),),
    mesh=_MESH,
    scratch_types=[
        pltpu.VMEM((8, CHUNK), jnp.int32),
        pltpu.VMEM((CHUNK, D), jnp.float32),
        pltpu.VMEM((CHUNK, D), jnp.float32),
        pltpu.VMEM_SHARED((N_PAD, D), jnp.float32),
    ])
def _sc_cnt(dst_hbm, zeros_hbm, ones_hbm, cnt_out,
            dst_v, onesrows, zbuf, acc_sh):
  c = lax.axis_index("c")
  s = lax.axis_index("s")
  wid = s * NC + c
  base = s * RPT

  pltpu.sync_copy(zeros_hbm, zbuf)
  pltpu.sync_copy(ones_hbm, onesrows)
  _acc_slice_copy(lambda r, sz: zbuf.at[pl.ds(0, sz)],
                  lambda r, sz: acc_sh.at[pl.ds(r, sz)], base)
  plsc.subcore_barrier()

  def chunk_step(t, carry):
    pltpu.sync_copy(dst_hbm.at[wid, pl.ds(t * 8, 8)], dst_v)
    for k in range(8):
      pltpu.sync_copy(onesrows, acc_sh.at[dst_v.at[k]], add=True)
    return carry
  lax.fori_loop(0, CHUNKS // 8, chunk_step, 0)

  plsc.subcore_barrier()

  _acc_slice_copy(lambda r, sz: acc_sh.at[pl.ds(r, sz)],
                  lambda r, sz: zbuf.at[pl.ds(0, sz)], base)
  _acc_slice_copy(lambda r, sz: zbuf.at[pl.ds(0, sz)],
                  lambda r, sz: cnt_out.at[c, pl.ds(r, sz)], base)


_BLK = 632


def _dot(a, b):
  return jnp.dot(a, b, preferred_element_type=jnp.float32)


def _proj_body(x_ref, w_ref, b_ref, o_ref):
  o_ref[...] = jnp.maximum(_dot(x_ref[...], w_ref[...]) + b_ref[...], 0.0)


def _proj(x, W, b):
  return pl.pallas_call(
      _proj_body,
      grid=(N_PAD // _BLK,),
      in_specs=[
          pl.BlockSpec((_BLK, D), lambda i: (i, 0)),
          pl.BlockSpec((D, D), lambda i: (0, 0)),
          pl.BlockSpec((D,), lambda i: (0,)),
      ],
      out_specs=pl.BlockSpec((_BLK, D), lambda i: (i, 0)),
      out_shape=jax.ShapeDtypeStruct((N_PAD, D), jnp.float32),
  )(x, W, b)


def _mean_agg(acc_ref, cnt_ref):
  ssum = acc_ref[0] + acc_ref[1]
  cnt = cnt_ref[0][:, 0:1] + cnt_ref[1][:, 0:1]
  return ssum / jnp.clip(cnt, 1.0, None)


def _combine1_body(acc_ref, cnt_ref, x_ref, wl_ref, bl_ref, wr_ref,
                   wp2_ref, bp2_ref, h_ref, xp2_ref):
  agg = _mean_agg(acc_ref, cnt_ref)
  h = jnp.maximum(
      _dot(agg, wl_ref[...]) + bl_ref[...] + _dot(x_ref[...], wr_ref[...]),
      0.0)
  h_ref[...] = h
  xp2_ref[...] = jnp.maximum(_dot(h, wp2_ref[...]) + bp2_ref[...], 0.0)


def _combine1(acc, cnt, x, Wl, bl, Wr, Wp2, bp2):
  return pl.pallas_call(
      _combine1_body,
      grid=(N_PAD // _BLK,),
      in_specs=[
          pl.BlockSpec((NC, _BLK, D), lambda i: (0, i, 0)),
          pl.BlockSpec((NC, _BLK, D), lambda i: (0, i, 0)),
          pl.BlockSpec((_BLK, D), lambda i: (i, 0)),
          pl.BlockSpec((D, D), lambda i: (0, 0)),
          pl.BlockSpec((D,), lambda i: (0,)),
          pl.BlockSpec((D, D), lambda i: (0, 0)),
          pl.BlockSpec((D, D), lambda i: (0, 0)),
          pl.BlockSpec((D,), lambda i: (0,)),
      ],
      out_specs=[
          pl.BlockSpec((_BLK, D), lambda i: (i, 0)),
          pl.BlockSpec((_BLK, D), lambda i: (i, 0)),
      ],
      out_shape=[
          jax.ShapeDtypeStruct((N_PAD, D), jnp.float32),
          jax.ShapeDtypeStruct((N_PAD, D), jnp.float32),
      ],
  )(acc, cnt, x, Wl, bl, Wr, Wp2, bp2)


def _combine2_body(acc_ref, cnt_ref, h_ref, wl_ref, bl_ref, wr_ref, o_ref):
  agg = _mean_agg(acc_ref, cnt_ref)
  o = (_dot(agg, wl_ref[...]) + bl_ref[...] + _dot(h_ref[...], wr_ref[...]))
  norm = jnp.sqrt(jnp.sum(o * o, axis=-1, keepdims=True))
  o = o / jnp.clip(norm, 1e-12, None)
  o = jnp.maximum(o, 0.0)
  m = jnp.max(o, axis=-1, keepdims=True)
  lse = m + jnp.log(jnp.sum(jnp.exp(o - m), axis=-1, keepdims=True))
  o_ref[...] = o - lse


def _combine2(acc, cnt, h, Wl, bl, Wr):
  return pl.pallas_call(
      _combine2_body,
      grid=(N_PAD // _BLK,),
      in_specs=[
          pl.BlockSpec((NC, _BLK, D), lambda i: (0, i, 0)),
          pl.BlockSpec((NC, _BLK, D), lambda i: (0, i, 0)),
          pl.BlockSpec((_BLK, D), lambda i: (i, 0)),
          pl.BlockSpec((D, D), lambda i: (0, 0)),
          pl.BlockSpec((D,), lambda i: (0,)),
          pl.BlockSpec((D, D), lambda i: (0, 0)),
      ],
      out_specs=pl.BlockSpec((_BLK, D), lambda i: (i, 0)),
      out_shape=jax.ShapeDtypeStruct((N_PAD, D), jnp.float32),
  )(acc, cnt, h, Wl, bl, Wr)


def kernel(matrix_nodes_features, edge_index, Wp1, bp1, Wl1, bl1, Wr1,
           Wp2, bp2, Wl2, bl2, Wr2):
  x = jnp.pad(matrix_nodes_features, ((0, N_PAD - N), (0, 0)))
  pad_i = np.arange(E_PAD - E, dtype=np.int32)
  pad_src = jnp.asarray((pad_i * 131) % N)
  pad_dst = jnp.asarray(N + pad_i % (N_PAD - N))
  src = jnp.concatenate([edge_index[0], pad_src]).reshape(NW, CHUNKS, CHUNK)
  dst = jnp.concatenate([edge_index[1], pad_dst]).reshape(NW, CHUNKS, CHUNK)
  zeros = jnp.zeros((CHUNK, D), jnp.float32)
  ones = jnp.ones((CHUNK, D), jnp.float32)

  (cnt,) = _sc_cnt(dst, zeros, ones)
  xp1 = _proj(x, Wp1, bp1)
  (acc1,) = _sc_agg(xp1, src, dst, zeros)
  h, xp2 = _combine1(acc1, cnt, x, Wl1, bl1, Wr1, Wp2, bp2)
  (acc2,) = _sc_agg(xp2, src, dst, zeros)
  out = _combine2(acc2, cnt, h, Wl2, bl2, Wr2)
  return out[:N]


# speedup vs baseline: 8.6283x; 1.0468x over previous
import functools

import numpy as np

import jax
import jax.numpy as jnp
from jax import lax
from jax.experimental import pallas as pl
from jax.experimental.pallas import tpu as pltpu
from jax.experimental.pallas import tpu_sc as plsc

N, E, D = 10000, 320000, 128
NC, NS = 2, 16
NW = NC * NS
CHUNK = 80
CHUNKS = 128
EPW = CHUNK * CHUNKS
E_PAD = EPW * NW
N_PAD = 10112
RPT = N_PAD // NS
_HIGH = lax.Precision.HIGHEST
_MESH = plsc.VectorSubcoreMesh(core_axis_name="c", subcore_axis_name="s")
_SLICES = tuple([CHUNK] * (RPT // CHUNK) +
                ([RPT % CHUNK] if RPT % CHUNK else []))


def _acc_slice_copy(src_at, dst_at, base):
  off = 0
  for sz in _SLICES:
    pltpu.sync_copy(src_at(base + off, sz), dst_at(base + off, sz))
    off += sz


@functools.partial(
    pl.kernel,
    out_type=(jax.ShapeDtypeStruct((NC, N_PAD, D), jnp.float32),),
    mesh=_MESH,
    scratch_types=[
        pltpu.VMEM((8, CHUNK), jnp.int32),
        pltpu.VMEM((8, CHUNK), jnp.int32),
        pltpu.VMEM((CHUNK, D), jnp.float32),
        pltpu.VMEM((CHUNK, D), jnp.float32),
        pltpu.VMEM_SHARED((N_PAD, D), jnp.float32),
        pltpu.SemaphoreType.DMA,
        pltpu.SemaphoreType.DMA,
    ])
def _sc_agg(xp_hbm, src_hbm, dst_hbm, zeros_hbm, acc_out,
            src_v, dst_v, rowbuf, rowbuf2, acc_sh, sem, sem2):
  c = lax.axis_index("c")
  s = lax.axis_index("s")
  wid = s * NC + c
  base = s * RPT
  rb = (rowbuf, rowbuf2)

  pltpu.sync_copy(zeros_hbm, rowbuf)
  _acc_slice_copy(lambda r, sz: rowbuf.at[pl.ds(0, sz)],
                  lambda r, sz: acc_sh.at[pl.ds(r, sz)], base)
  plsc.subcore_barrier()

  def chunk_step(t, carry):
    pltpu.sync_copy(src_hbm.at[wid, pl.ds(t * 8, 8)], src_v)
    pltpu.sync_copy(dst_hbm.at[wid, pl.ds(t * 8, 8)], dst_v)
    pltpu.async_copy(xp_hbm.at[src_v.at[0]], rb[0], sem)
    for k in range(8):
      if k + 1 < 8:
        if k >= 1:
          pltpu.make_async_copy(rb[(k - 1) % 2], acc_sh.at[dst_v.at[k - 1]],
                                sem2).wait()
        pltpu.async_copy(xp_hbm.at[src_v.at[k + 1]], rb[(k + 1) % 2], sem)
      pltpu.make_async_copy(xp_hbm.at[src_v.at[k]], rb[k % 2], sem).wait()
      pltpu.async_copy(rb[k % 2], acc_sh.at[dst_v.at[k]], sem2, add=True)
    pltpu.make_async_copy(rb[0], acc_sh.at[dst_v.at[6]], sem2).wait()
    pltpu.make_async_copy(rb[1], acc_sh.at[dst_v.at[7]], sem2).wait()
    return carry
  lax.fori_loop(0, CHUNKS // 8, chunk_step, 0)

  plsc.subcore_barrier()

  _acc_slice_copy(lambda r, sz: acc_sh.at[pl.ds(r, sz)],
                  lambda r, sz: rowbuf.at[pl.ds(0, sz)], base)
  _acc_slice_copy(lambda r, sz: rowbuf.at[pl.ds(0, sz)],
                  lambda r, sz: acc_out.at[c, pl.ds(r, sz)], base)


@functools.partial(
    pl.kernel,
    out_type=(jax.ShapeDtypeStruct((NC, N_PAD, D), jnp.float32),),
    mesh=_MESH,
    scratch_types=[
        pltpu.VMEM((8, CHUNK), jnp.int32),
        pltpu.VMEM((CHUNK, D), jnp.float32),
        pltpu.VMEM((CHUNK, D), jnp.float32),
        pltpu.VMEM_SHARED((N_PAD, D), jnp.float32),
    ])
def _sc_cnt(dst_hbm, zeros_hbm, ones_hbm, cnt_out,
            dst_v, onesrows, zbuf, acc_sh):
  c = lax.axis_index("c")
  s = lax.axis_index("s")
  wid = s * NC + c
  base = s * RPT

  pltpu.sync_copy(zeros_hbm, zbuf)
  pltpu.sync_copy(ones_hbm, onesrows)
  _acc_slice_copy(lambda r, sz: zbuf.at[pl.ds(0, sz)],
                  lambda r, sz: acc_sh.at[pl.ds(r, sz)], base)
  plsc.subcore_barrier()

  def chunk_step(t, carry):
    pltpu.sync_copy(dst_hbm.at[wid, pl.ds(t * 8, 8)], dst_v)
    for k in range(8):
      pltpu.sync_copy(onesrows, acc_sh.at[dst_v.at[k]], add=True)
    return carry
  lax.fori_loop(0, CHUNKS // 8, chunk_step, 0)

  plsc.subcore_barrier()

  _acc_slice_copy(lambda r, sz: acc_sh.at[pl.ds(r, sz)],
                  lambda r, sz: zbuf.at[pl.ds(0, sz)], base)
  _acc_slice_copy(lambda r, sz: zbuf.at[pl.ds(0, sz)],
                  lambda r, sz: cnt_out.at[c, pl.ds(r, sz)], base)


_BLK = 2528


def _dot(a, b):
  return jnp.dot(a, b, preferred_element_type=jnp.float32)


def _proj_body(x_ref, w_ref, b_ref, o_ref):
  o_ref[...] = jnp.maximum(_dot(x_ref[...], w_ref[...]) + b_ref[...], 0.0)


def _proj(x, W, b):
  return pl.pallas_call(
      _proj_body,
      grid=(N_PAD // _BLK,),
      in_specs=[
          pl.BlockSpec((_BLK, D), lambda i: (i, 0)),
          pl.BlockSpec((D, D), lambda i: (0, 0)),
          pl.BlockSpec((D,), lambda i: (0,)),
      ],
      out_specs=pl.BlockSpec((_BLK, D), lambda i: (i, 0)),
      out_shape=jax.ShapeDtypeStruct((N_PAD, D), jnp.float32),
  )(x, W, b)


def _mean_agg(acc_ref, cnt_ref):
  ssum = acc_ref[0] + acc_ref[1]
  cnt = cnt_ref[0][:, 0:1] + cnt_ref[1][:, 0:1]
  return ssum / jnp.clip(cnt, 1.0, None)


def _combine1_body(acc_ref, cnt_ref, x_ref, wl_ref, bl_ref, wr_ref,
                   wp2_ref, bp2_ref, h_ref, xp2_ref):
  agg = _mean_agg(acc_ref, cnt_ref)
  h = jnp.maximum(
      _dot(agg, wl_ref[...]) + bl_ref[...] + _dot(x_ref[...], wr_ref[...]),
      0.0)
  h_ref[...] = h
  xp2_ref[...] = jnp.maximum(_dot(h, wp2_ref[...]) + bp2_ref[...], 0.0)


def _combine1(acc, cnt, x, Wl, bl, Wr, Wp2, bp2):
  return pl.pallas_call(
      _combine1_body,
      grid=(N_PAD // _BLK,),
      in_specs=[
          pl.BlockSpec((NC, _BLK, D), lambda i: (0, i, 0)),
          pl.BlockSpec((NC, _BLK, D), lambda i: (0, i, 0)),
          pl.BlockSpec((_BLK, D), lambda i: (i, 0)),
          pl.BlockSpec((D, D), lambda i: (0, 0)),
          pl.BlockSpec((D,), lambda i: (0,)),
          pl.BlockSpec((D, D), lambda i: (0, 0)),
          pl.BlockSpec((D, D), lambda i: (0, 0)),
          pl.BlockSpec((D,), lambda i: (0,)),
      ],
      out_specs=[
          pl.BlockSpec((_BLK, D), lambda i: (i, 0)),
          pl.BlockSpec((_BLK, D), lambda i: (i, 0)),
      ],
      out_shape=[
          jax.ShapeDtypeStruct((N_PAD, D), jnp.float32),
          jax.ShapeDtypeStruct((N_PAD, D), jnp.float32),
      ],
  )(acc, cnt, x, Wl, bl, Wr, Wp2, bp2)


def _combine2_body(acc_ref, cnt_ref, h_ref, wl_ref, bl_ref, wr_ref, o_ref):
  agg = _mean_agg(acc_ref, cnt_ref)
  o = (_dot(agg, wl_ref[...]) + bl_ref[...] + _dot(h_ref[...], wr_ref[...]))
  norm = jnp.sqrt(jnp.sum(o * o, axis=-1, keepdims=True))
  o = o / jnp.clip(norm, 1e-12, None)
  o = jnp.maximum(o, 0.0)
  m = jnp.max(o, axis=-1, keepdims=True)
  lse = m + jnp.log(jnp.sum(jnp.exp(o - m), axis=-1, keepdims=True))
  o_ref[...] = o - lse


def _combine2(acc, cnt, h, Wl, bl, Wr):
  return pl.pallas_call(
      _combine2_body,
      grid=(N_PAD // _BLK,),
      in_specs=[
          pl.BlockSpec((NC, _BLK, D), lambda i: (0, i, 0)),
          pl.BlockSpec((NC, _BLK, D), lambda i: (0, i, 0)),
          pl.BlockSpec((_BLK, D), lambda i: (i, 0)),
          pl.BlockSpec((D, D), lambda i: (0, 0)),
          pl.BlockSpec((D,), lambda i: (0,)),
          pl.BlockSpec((D, D), lambda i: (0, 0)),
      ],
      out_specs=pl.BlockSpec((_BLK, D), lambda i: (i, 0)),
      out_shape=jax.ShapeDtypeStruct((N_PAD, D), jnp.float32),
  )(acc, cnt, h, Wl, bl, Wr)


def kernel(matrix_nodes_features, edge_index, Wp1, bp1, Wl1, bl1, Wr1,
           Wp2, bp2, Wl2, bl2, Wr2):
  x = jnp.pad(matrix_nodes_features, ((0, N_PAD - N), (0, 0)))
  pad_i = np.arange(E_PAD - E, dtype=np.int32)
  pad_src = jnp.asarray((pad_i * 131) % N)
  pad_dst = jnp.asarray(N + pad_i % (N_PAD - N))
  src = jnp.concatenate([edge_index[0], pad_src]).reshape(NW, CHUNKS, CHUNK)
  dst = jnp.concatenate([edge_index[1], pad_dst]).reshape(NW, CHUNKS, CHUNK)
  zeros = jnp.zeros((CHUNK, D), jnp.float32)
  ones = jnp.ones((CHUNK, D), jnp.float32)

  (cnt,) = _sc_cnt(dst, zeros, ones)
  xp1 = _proj(x, Wp1, bp1)
  (acc1,) = _sc_agg(xp1, src, dst, zeros)
  h, xp2 = _combine1(acc1, cnt, x, Wl1, bl1, Wr1, Wp2, bp2)
  (acc2,) = _sc_agg(xp2, src, dst, zeros)
  out = _combine2(acc2, cnt, h, Wl2, bl2, Wr2)
  return out[:N]
